# baseline (speedup 1.0000x reference)
import jax
import jax.numpy as jnp
from jax.experimental import pallas as pl
from jax.experimental.pallas import tpu as pltpu

L_COORD, L_OBJ, L_NOOBJ = 5.0, 1.0, 0.5
CCHUNK = 16


def _body(pc, pr, pb, lc, lr, lb, out_ref):
    b = pl.program_id(0)
    c = pl.program_id(1)

    pc_, lc_ = pc[0], lc[0]
    pr_, lr_ = pr[0], lr[0]
    pb_, lb_ = pb[0], lb[0]

    valid = (lr_[0:1] + lr_[1:2] > 0.9).astype(jnp.float32)

    cls_p = jnp.sum(((pc_ - lc_) ** 2) * valid)

    @pl.when(jnp.logical_and(b == 0, c == 0))
    def _():
        out_ref[...] = jnp.zeros_like(out_ref)

    out_ref[2:3, :] += jnp.full((1, 128), cls_p, jnp.float32)

    @pl.when(c == 0)
    def _():
        neg = jnp.sum(((pr_ - lr_) ** 2) * (lr_ < 1.0).astype(jnp.float32))

        def corners(o):
            x1 = o[0:1] - o[2:3] * 0.5
            y1 = o[1:2] - o[3:4] * 0.5
            return x1, y1, x1 + o[2:3], y1 + o[3:4]

        def iou(b1, b2):
            lx = jnp.maximum(b1[0], b2[0])
            ly = jnp.maximum(b1[1], b2[1])
            rx = jnp.minimum(b1[2], b2[2])
            ry = jnp.minimum(b1[3], b2[3])
            inter = jnp.maximum(rx - lx, 0.0) * jnp.maximum(ry - ly, 0.0)
            a1 = (b1[2] - b1[0]) * (b1[3] - b1[1])
            a2 = (b2[2] - b2[0]) * (b2[3] - b2[1])
            return inter / (a1 + a2 - inter + 0.0001)

        iou0 = iou(corners(lb_[0:4]), corners(pb_[0:4]))
        iou1 = iou(corners(lb_[4:8]), corners(pb_[4:8]))
        sel = iou1 > iou0
        best_iou = jnp.where(sel, iou1, iou0)
        best_pr = jnp.where(sel, pr_[1:2], pr_[0:1])
        pobj = jnp.sum(((best_pr - best_iou) ** 2) * valid)

        d = (pb_ - lb_) ** 2
        off0 = d[0:1] + d[1:2] + d[2:3] + d[3:4]
        off1 = d[4:5] + d[5:6] + d[6:7] + d[7:8]
        off = jnp.sum(jnp.where(sel, off1, off0) * valid)

        out_ref[0:1, :] += jnp.full((1, 128), pobj, jnp.float32)
        out_ref[1:2, :] += jnp.full((1, 128), neg, jnp.float32)
        out_ref[3:4, :] += jnp.full((1, 128), off, jnp.float32)


def kernel(pred_cls, pred_response, pred_bboxes, label_cls, label_response, label_bboxes):
    B, CLS, H, W = pred_cls.shape
    BB = pred_response.shape[1]
    NC = CLS // CCHUNK

    acc = pl.pallas_call(
        _body,
        grid=(B, NC),
        in_specs=[
            pl.BlockSpec((1, CCHUNK, H, W), lambda b, c: (b, c, 0, 0)),
            pl.BlockSpec((1, BB, H, W), lambda b, c: (b, 0, 0, 0)),
            pl.BlockSpec((1, BB * 4, H, W), lambda b, c: (b, 0, 0, 0)),
            pl.BlockSpec((1, CCHUNK, H, W), lambda b, c: (b, c, 0, 0)),
            pl.BlockSpec((1, BB, H, W), lambda b, c: (b, 0, 0, 0)),
            pl.BlockSpec((1, BB * 4, H, W), lambda b, c: (b, 0, 0, 0)),
        ],
        out_specs=pl.BlockSpec((4, 128), lambda b, c: (0, 0)),
        out_shape=jax.ShapeDtypeStruct((4, 128), jnp.float32),
    )(pred_cls, pred_response, pred_bboxes, label_cls, label_response, label_bboxes)

    inv_b = 1.0 / B
    return {"pObj": acc[0, 0] * (inv_b * L_OBJ),
            "nObj": acc[1, 0] * (inv_b * L_NOOBJ),
            "cls": acc[2, 0] * inv_b,
            "offset": acc[3, 0] * (inv_b * L_COORD)}


# speedup vs baseline: 1.1557x; 1.1557x over previous
import functools

import jax
import jax.numpy as jnp
from jax import lax
from jax.experimental import pallas as pl
from jax.experimental.pallas import tpu as pltpu
from jax.experimental.pallas import tpu_sc as plsc

L_COORD, L_OBJ, L_NOOBJ = 5.0, 1.0, 0.5
NCORE, NSUB, LANES = 2, 16, 16
NW = NCORE * NSUB
CC = 4


def _sc_body(shapes):
    B, CLS, BB, HW = shapes
    BPW = B // NW
    NJ = HW // LANES
    NCH = CLS // CC

    def body(pc, lc, pr, lr, pb, lb, out,
             lrb, prb, pbb, lbb, validv, pcb0, pcb1, lcb0, lcb1, outv,
             sp0, sp1, sl0, sl1):
        wid = lax.axis_index("c") * NSUB + lax.axis_index("s")

        def iou(tx1, ty1, tx2, ty2, qx1, qy1, qx2, qy2):
            ix1 = jnp.maximum(tx1, qx1)
            iy1 = jnp.maximum(ty1, qy1)
            ix2 = jnp.minimum(tx2, qx2)
            iy2 = jnp.minimum(ty2, qy2)
            inter = jnp.maximum(ix2 - ix1, 0.0) * jnp.maximum(iy2 - iy1, 0.0)
            a1 = (tx2 - tx1) * (ty2 - ty1)
            a2 = (qx2 - qx1) * (qy2 - qy1)
            return inter / (a1 + a2 - inter + 0.0001)

        def corners(x, y, w, h):
            x1 = x - w * 0.5
            y1 = y - h * 0.5
            return x1, y1, x1 + w, y1 + h

        def s1_body(j, carry):
            neg, pobj, off = carry
            s = pl.ds(j * LANES, LANES)
            lr0 = lrb[0, s]
            lr1 = lrb[1, s]
            pr0 = prb[0, s]
            pr1 = prb[1, s]
            valid = jnp.where(lr0 + lr1 > 0.9, 1.0, 0.0).astype(jnp.float32)
            validv[s] = valid
            d0 = pr0 - lr0
            d1 = pr1 - lr1
            neg = (neg + d0 * d0 * jnp.where(lr0 < 1.0, 1.0, 0.0)
                   + d1 * d1 * jnp.where(lr1 < 1.0, 1.0, 0.0))

            lx0, ly0, lw0, lh0 = lbb[0, s], lbb[1, s], lbb[2, s], lbb[3, s]
            lx1, ly1, lw1, lh1 = lbb[4, s], lbb[5, s], lbb[6, s], lbb[7, s]
            px0, py0, pw0, ph0 = pbb[0, s], pbb[1, s], pbb[2, s], pbb[3, s]
            px1, py1, pw1, ph1 = pbb[4, s], pbb[5, s], pbb[6, s], pbb[7, s]

            iou0 = iou(*corners(lx0, ly0, lw0, lh0), *corners(px0, py0, pw0, ph0))
            iou1 = iou(*corners(lx1, ly1, lw1, lh1), *corners(px1, py1, pw1, ph1))
            sel = iou1 > iou0
            best_iou = jnp.where(sel, iou1, iou0)
            best_pr = jnp.where(sel, pr1, pr0)
            dr = best_pr - best_iou
            pobj = pobj + dr * dr * valid

            ex0, ey0, ew0, eh0 = px0 - lx0, py0 - ly0, pw0 - lw0, ph0 - lh0
            ex1, ey1, ew1, eh1 = px1 - lx1, py1 - ly1, pw1 - lw1, ph1 - lh1
            off0 = ex0 * ex0 + ey0 * ey0 + ew0 * ew0 + eh0 * eh0
            off1 = ex1 * ex1 + ey1 * ey1 + ew1 * ew1 + eh1 * eh1
            off = off + jnp.where(sel, off1, off0) * valid
            return neg, pobj, off

        def make_s2_body(bp, bl):
            def s2_body(j, acc):
                s = pl.ds(j * LANES, LANES)
                v = validv[s]
                t = None
                for r in range(CC):
                    d = bp[r, s] - bl[r, s]
                    t = d * d if t is None else t + d * d
                return acc + t * v
            return s2_body

        zeros = jnp.zeros((LANES,), jnp.float32)
        neg, pobj, off, cls_a = zeros, zeros, zeros, zeros
        bufs = ((pcb0, lcb0, sp0, sl0), (pcb1, lcb1, sp1, sl1))

        for bi in range(BPW):
            b = wid * BPW + bi
            pltpu.sync_copy(lr.at[pl.ds(b * BB, BB), :], lrb)
            pltpu.sync_copy(pr.at[pl.ds(b * BB, BB), :], prb)
            pltpu.sync_copy(pb.at[pl.ds(b * BB * 4, BB * 4), :], pbb)
            pltpu.sync_copy(lb.at[pl.ds(b * BB * 4, BB * 4), :], lbb)
            neg, pobj, off = lax.fori_loop(0, NJ, s1_body, (neg, pobj, off))

            def issue(ci, slot):
                bp, bl, sp, sl = bufs[slot]
                r0 = b * CLS + ci * CC
                cpp = pltpu.async_copy(pc.at[pl.ds(r0, CC), :], bp, sp)
                cpl = pltpu.async_copy(lc.at[pl.ds(r0, CC), :], bl, sl)
                return cpp, cpl

            pend = issue(0, 0)
            for ci in range(NCH):
                slot = ci % 2
                nxt = None
                if ci + 1 < NCH:
                    nxt = issue(ci + 1, 1 - slot)
                pend[0].wait()
                pend[1].wait()
                bp, bl = bufs[slot][0], bufs[slot][1]
                cls_a = lax.fori_loop(0, NJ, make_s2_body(bp, bl), cls_a)
                pend = nxt

        outv[0, :] = pobj
        outv[1, :] = neg
        outv[2, :] = cls_a
        outv[3, :] = off
        pltpu.sync_copy(outv, out.at[wid])

    return body


def kernel(pred_cls, pred_response, pred_bboxes, label_cls, label_response, label_bboxes):
    B, CLS, H, W = pred_cls.shape
    BB = pred_response.shape[1]
    HW = H * W

    pc = pred_cls.reshape(B * CLS, HW)
    lc = label_cls.reshape(B * CLS, HW)
    pr = pred_response.reshape(B * BB, HW)
    lr = label_response.reshape(B * BB, HW)
    pb = pred_bboxes.reshape(B * BB * 4, HW)
    lb = label_bboxes.reshape(B * BB * 4, HW)

    mesh = plsc.VectorSubcoreMesh(core_axis_name="c", subcore_axis_name="s")
    f32 = jnp.float32
    run = pl.kernel(
        _sc_body((B, CLS, BB, HW)),
        out_type=jax.ShapeDtypeStruct((NW, 4, LANES), f32),
        mesh=mesh,
        scratch_types=[
            pltpu.VMEM((BB, HW), f32),
            pltpu.VMEM((BB, HW), f32),
            pltpu.VMEM((BB * 4, HW), f32),
            pltpu.VMEM((BB * 4, HW), f32),
            pltpu.VMEM((HW,), f32),
            pltpu.VMEM((CC, HW), f32),
            pltpu.VMEM((CC, HW), f32),
            pltpu.VMEM((CC, HW), f32),
            pltpu.VMEM((CC, HW), f32),
            pltpu.VMEM((4, LANES), f32),
            pltpu.SemaphoreType.DMA,
            pltpu.SemaphoreType.DMA,
            pltpu.SemaphoreType.DMA,
            pltpu.SemaphoreType.DMA,
        ],
    )
    acc = run(pc, lc, pr, lr, pb, lb)
    sums = jnp.sum(acc, axis=(0, 2))
    inv_b = 1.0 / B
    return {"pObj": sums[0] * (inv_b * L_OBJ),
            "nObj": sums[1] * (inv_b * L_NOOBJ),
            "cls": sums[2] * inv_b,
            "offset": sums[3] * (inv_b * L_COORD)}


# speedup vs baseline: 1.2194x; 1.0551x over previous
import jax
import jax.numpy as jnp
from jax import lax
from jax.experimental import pallas as pl
from jax.experimental.pallas import tpu as pltpu
from jax.experimental.pallas import tpu_sc as plsc

L_COORD, L_OBJ, L_NOOBJ = 5.0, 1.0, 0.5
NCORE, NSUB, LANES = 2, 16, 16
NW = NCORE * NSUB
CC = 2
QROWS = 8
XOFF = (0, 16, 32, 40)
VOFF = (0, 16, 32, 48)


def _sc_body(shapes):
    B, CLS, BB, H, W = shapes
    BPW = B // NW
    NCH = CLS // CC
    NPAIR = NCH // 2
    NQ = H // QROWS

    def body(pc, pr, pb, lc, lr, lb, out,
             lrb0, lrb1, prb0, prb1, pbb0, pbb1, lbb0, lbb1, validv,
             pcb0, pcb1, lcb0, lcb1, outv,
             sp0, sp1, sl0, sl1, sb0, sb1):
        wid = lax.axis_index("c") * NSUB + lax.axis_index("s")
        m3 = jnp.where(lax.broadcasted_iota(jnp.int32, (LANES,), 0) >= 8,
                       1.0, 0.0).astype(jnp.float32)

        def iou(tx1, ty1, tx2, ty2, qx1, qy1, qx2, qy2):
            ix1 = jnp.maximum(tx1, qx1)
            iy1 = jnp.maximum(ty1, qy1)
            ix2 = jnp.minimum(tx2, qx2)
            iy2 = jnp.minimum(ty2, qy2)
            inter = jnp.maximum(ix2 - ix1, 0.0) * jnp.maximum(iy2 - iy1, 0.0)
            a1 = (tx2 - tx1) * (ty2 - ty1)
            a2 = (qx2 - qx1) * (qy2 - qy1)
            return inter / (a1 + a2 - inter + 0.0001)

        def corners(x, y, w, h):
            x1 = x - w * 0.5
            y1 = y - h * 0.5
            return x1, y1, x1 + w, y1 + h

        zeros = jnp.zeros((LANES,), jnp.float32)
        neg, pobj, off, cls_a = zeros, zeros, zeros, zeros
        cbufs = ((pcb0, lcb0, sp0, sl0), (pcb1, lcb1, sp1, sl1))
        bbufs = ((lrb0, prb0, pbb0, lbb0, sb0), (lrb1, prb1, pbb1, lbb1, sb1))

        for bi in range(BPW):
            b = wid * BPW + bi

            def issue_cls(ci, slot):
                bp, bl, sp, sl = cbufs[slot]
                c0 = ci * CC
                pltpu.async_copy(pc.at[b, pl.ds(c0, CC), :, :], bp, sp)
                pltpu.async_copy(lc.at[b, pl.ds(c0, CC), :, :], bl, sl)

            def wait_cls(slot):
                bp, bl, sp, sl = cbufs[slot]
                pltpu.make_async_copy(pc.at[b, pl.ds(0, CC), :, :], bp, sp).wait()
                pltpu.make_async_copy(lc.at[b, pl.ds(0, CC), :, :], bl, sl).wait()

            def issue_box(q, slot):
                blr, bpr, bp, bl, sb = bbufs[slot]
                r0 = q * QROWS
                pltpu.async_copy(lr.at[b, :, pl.ds(r0, QROWS), :], blr, sb)
                pltpu.async_copy(pr.at[b, :, pl.ds(r0, QROWS), :], bpr, sb)
                pltpu.async_copy(pb.at[b, :, pl.ds(r0, QROWS), :], bp, sb)
                pltpu.async_copy(lb.at[b, :, pl.ds(r0, QROWS), :], bl, sb)

            def wait_box(slot):
                blr, bpr, bp, bl, sb = bbufs[slot]
                pltpu.make_async_copy(lr.at[b, :, pl.ds(0, QROWS), :], blr, sb).wait()
                pltpu.make_async_copy(pr.at[b, :, pl.ds(0, QROWS), :], bpr, sb).wait()
                pltpu.make_async_copy(pb.at[b, :, pl.ds(0, QROWS), :], bp, sb).wait()
                pltpu.make_async_copy(lb.at[b, :, pl.ds(0, QROWS), :], bl, sb).wait()

            issue_cls(0, 0)
            issue_box(0, 0)

            for q in range(NQ):
                slot = q % 2
                if q + 1 < NQ:
                    issue_box(q + 1, 1 - slot)
                wait_box(slot)
                lrb, prb, pbb, lbb = (bbufs[slot][0], bbufs[slot][1],
                                      bbufs[slot][2], bbufs[slot][3])

                def box_body(t, carry, lrb=lrb, prb=prb, pbb=pbb, lbb=lbb, q=q):
                    neg, pobj, off = carry
                    y = lax.shift_right_logical(t, 2)
                    j = lax.bitwise_and(t, 3)
                    is_tail = j == 3
                    xoff = jnp.where(is_tail, 40, j * LANES)
                    s = pl.ds(xoff, LANES)
                    vy = q * QROWS + y
                    mj = jnp.where(is_tail, m3, 1.0).astype(jnp.float32)
                    lr0 = lrb[0, y, s]
                    lr1 = lrb[1, y, s]
                    pr0 = prb[0, y, s]
                    pr1 = prb[1, y, s]
                    valid = jnp.where(lr0 + lr1 > 0.9, mj, 0.0)
                    negj = ((pr0 - lr0) * (pr0 - lr0) * jnp.where(lr0 < 1.0, mj, 0.0)
                            + (pr1 - lr1) * (pr1 - lr1) * jnp.where(lr1 < 1.0, mj, 0.0))
                    validv[vy, pl.ds(j * LANES, LANES)] = valid
                    neg = neg + negj
                    lx0, ly0, lw0, lh0 = lbb[0, y, s], lbb[1, y, s], lbb[2, y, s], lbb[3, y, s]
                    lx1, ly1, lw1, lh1 = lbb[4, y, s], lbb[5, y, s], lbb[6, y, s], lbb[7, y, s]
                    px0, py0, pw0, ph0 = pbb[0, y, s], pbb[1, y, s], pbb[2, y, s], pbb[3, y, s]
                    px1, py1, pw1, ph1 = pbb[4, y, s], pbb[5, y, s], pbb[6, y, s], pbb[7, y, s]
                    iou0 = iou(*corners(lx0, ly0, lw0, lh0), *corners(px0, py0, pw0, ph0))
                    iou1 = iou(*corners(lx1, ly1, lw1, lh1), *corners(px1, py1, pw1, ph1))
                    sel = iou1 > iou0
                    best_iou = jnp.where(sel, iou1, iou0)
                    best_pr = jnp.where(sel, pr1, pr0)
                    dr = best_pr - best_iou
                    pobj = pobj + dr * dr * valid
                    ex0, ey0, ew0, eh0 = px0 - lx0, py0 - ly0, pw0 - lw0, ph0 - lh0
                    ex1, ey1, ew1, eh1 = px1 - lx1, py1 - ly1, pw1 - lw1, ph1 - lh1
                    off0 = ex0 * ex0 + ey0 * ey0 + ew0 * ew0 + eh0 * eh0
                    off1 = ex1 * ex1 + ey1 * ey1 + ew1 * ew1 + eh1 * eh1
                    off = off + jnp.where(sel, off1, off0) * valid
                    return neg, pobj, off

                neg, pobj, off = lax.fori_loop(0, QROWS * 4, box_body,
                                               (neg, pobj, off))

            def make_cls_body(bp, bl):
                def cbody(t, acc):
                    y = lax.shift_right_logical(t, 2)
                    j = lax.bitwise_and(t, 3)
                    xoff = jnp.where(j == 3, 40, j * LANES)
                    s = pl.ds(xoff, LANES)
                    v = validv[y, pl.ds(j * LANES, LANES)]
                    tt = None
                    for ch in range(CC):
                        d = bp[ch, y, s] - bl[ch, y, s]
                        tt = d * d if tt is None else tt + d * d
                    return acc + tt * v
                return cbody

            def pair_body(k, acc):
                issue_cls(2 * k + 1, 1)
                wait_cls(0)
                acc = lax.fori_loop(0, H * 4, make_cls_body(pcb0, lcb0), acc)
                nxt = 2 * k + 2
                issue_cls(jnp.where(nxt >= NCH, 0, nxt), 0)
                wait_cls(1)
                acc = lax.fori_loop(0, H * 4, make_cls_body(pcb1, lcb1), acc)
                return acc

            cls_a = lax.fori_loop(0, NPAIR, pair_body, cls_a)
            wait_cls(0)

        outv[0, :] = pobj
        outv[1, :] = neg
        outv[2, :] = cls_a
        outv[3, :] = off
        pltpu.sync_copy(outv, out.at[wid])

    return body


def kernel(pred_cls, pred_response, pred_bboxes, label_cls, label_response, label_bboxes):
    B, CLS, H, W = pred_cls.shape
    BB = pred_response.shape[1]

    mesh = plsc.VectorSubcoreMesh(core_axis_name="c", subcore_axis_name="s")
    f32 = jnp.float32
    run = pl.kernel(
        _sc_body((B, CLS, BB, H, W)),
        out_type=jax.ShapeDtypeStruct((NW, 4, LANES), f32),
        mesh=mesh,
        scratch_types=[
            pltpu.VMEM((BB, QROWS, W), f32),
            pltpu.VMEM((BB, QROWS, W), f32),
            pltpu.VMEM((BB, QROWS, W), f32),
            pltpu.VMEM((BB, QROWS, W), f32),
            pltpu.VMEM((BB * 4, QROWS, W), f32),
            pltpu.VMEM((BB * 4, QROWS, W), f32),
            pltpu.VMEM((BB * 4, QROWS, W), f32),
            pltpu.VMEM((BB * 4, QROWS, W), f32),
            pltpu.VMEM((H, 64), f32),
            pltpu.VMEM((CC, H, W), f32),
            pltpu.VMEM((CC, H, W), f32),
            pltpu.VMEM((CC, H, W), f32),
            pltpu.VMEM((CC, H, W), f32),
            pltpu.VMEM((4, LANES), f32),
            pltpu.SemaphoreType.DMA,
            pltpu.SemaphoreType.DMA,
            pltpu.SemaphoreType.DMA,
            pltpu.SemaphoreType.DMA,
            pltpu.SemaphoreType.DMA,
            pltpu.SemaphoreType.DMA,
        ],
    )
    acc = run(pred_cls, pred_response, pred_bboxes, label_cls, label_response, label_bboxes)
    sums = jnp.sum(acc, axis=(0, 2))
    inv_b = 1.0 / B
    return {"pObj": sums[0] * (inv_b * L_OBJ),
            "nObj": sums[1] * (inv_b * L_NOOBJ),
            "cls": sums[2] * inv_b,
            "offset": sums[3] * (inv_b * L_COORD)}


# speedup vs baseline: 1.7599x; 1.4433x over previous
import jax
import jax.numpy as jnp
from jax.experimental import pallas as pl
from jax.experimental.pallas import tpu as pltpu

L_COORD, L_OBJ, L_NOOBJ = 5.0, 1.0, 0.5


def _body(pc, pr, pb, lc, lr, lb, out_ref):
    b = pl.program_id(0)
    pc_ = pc[0].astype(jnp.float32)
    lc_ = lc[0].astype(jnp.float32)
    pr_, lr_ = pr[0], lr[0]
    pb_, lb_ = pb[0], lb[0]

    valid = (lr_[0:1] + lr_[1:2] > 0.9).astype(jnp.float32)

    cls_p = jnp.sum(((pc_ - lc_) ** 2) * valid)
    neg = jnp.sum(((pr_ - lr_) ** 2) * (lr_ < 1.0).astype(jnp.float32))

    def corners(o):
        x1 = o[0:1] - o[2:3] * 0.5
        y1 = o[1:2] - o[3:4] * 0.5
        return x1, y1, x1 + o[2:3], y1 + o[3:4]

    def iou(b1, b2):
        lx = jnp.maximum(b1[0], b2[0])
        ly = jnp.maximum(b1[1], b2[1])
        rx = jnp.minimum(b1[2], b2[2])
        ry = jnp.minimum(b1[3], b2[3])
        inter = jnp.maximum(rx - lx, 0.0) * jnp.maximum(ry - ly, 0.0)
        a1 = (b1[2] - b1[0]) * (b1[3] - b1[1])
        a2 = (b2[2] - b2[0]) * (b2[3] - b2[1])
        return inter / (a1 + a2 - inter + 0.0001)

    iou0 = iou(corners(lb_[0:4]), corners(pb_[0:4]))
    iou1 = iou(corners(lb_[4:8]), corners(pb_[4:8]))
    sel = iou1 > iou0
    best_iou = jnp.where(sel, iou1, iou0)
    best_pr = jnp.where(sel, pr_[1:2], pr_[0:1])
    pobj = jnp.sum(((best_pr - best_iou) ** 2) * valid)

    d = (pb_ - lb_) ** 2
    off0 = d[0:1] + d[1:2] + d[2:3] + d[3:4]
    off1 = d[4:5] + d[5:6] + d[6:7] + d[7:8]
    off = jnp.sum(jnp.where(sel, off1, off0) * valid)

    part = jnp.concatenate(
        [jnp.full((1, 128), pobj, jnp.float32),
         jnp.full((1, 128), neg, jnp.float32),
         jnp.full((1, 128), cls_p, jnp.float32),
         jnp.full((1, 128), off, jnp.float32)], axis=0)

    @pl.when(b == 0)
    def _():
        out_ref[...] = jnp.zeros_like(out_ref)

    out_ref[...] += part


def kernel(pred_cls, pred_response, pred_bboxes, label_cls, label_response, label_bboxes):
    B, CLS, H, W = pred_cls.shape
    BB = pred_response.shape[1]
    HW = H * W
    pc = pred_cls.reshape(B, CLS, HW).astype(jnp.bfloat16)
    lc = label_cls.reshape(B, CLS, HW).astype(jnp.bfloat16)
    pr = pred_response.reshape(B, BB, HW)
    lr = label_response.reshape(B, BB, HW)
    pb = pred_bboxes.reshape(B, BB * 4, HW)
    lb = label_bboxes.reshape(B, BB * 4, HW)

    acc = pl.pallas_call(
        _body,
        grid=(B,),
        in_specs=[
            pl.BlockSpec((1, CLS, HW), lambda b: (b, 0, 0)),
            pl.BlockSpec((1, BB, HW), lambda b: (b, 0, 0)),
            pl.BlockSpec((1, BB * 4, HW), lambda b: (b, 0, 0)),
            pl.BlockSpec((1, CLS, HW), lambda b: (b, 0, 0)),
            pl.BlockSpec((1, BB, HW), lambda b: (b, 0, 0)),
            pl.BlockSpec((1, BB * 4, HW), lambda b: (b, 0, 0)),
        ],
        out_specs=pl.BlockSpec((4, 128), lambda b: (0, 0)),
        out_shape=jax.ShapeDtypeStruct((4, 128), jnp.float32),
    )(pc, pr, pb, lc, lr, lb)

    inv_b = 1.0 / B
    return {"pObj": acc[0, 0] * (inv_b * L_OBJ),
            "nObj": acc[1, 0] * (inv_b * L_NOOBJ),
            "cls": acc[2, 0] * inv_b,
            "offset": acc[3, 0] * (inv_b * L_COORD)}


# speedup vs baseline: 1.8248x; 1.0369x over previous
import jax
import jax.numpy as jnp
from jax import lax
from jax.experimental import pallas as pl
from jax.experimental.pallas import tpu as pltpu
from jax.experimental.pallas import tpu_sc as plsc

L_COORD, L_OBJ, L_NOOBJ = 5.0, 1.0, 0.5
NCORE, NSUB, LANES = 2, 16, 16
NW = NCORE * NSUB
QROWS = 8


def _sc_body(shapes):
    B, BB, H, W = shapes
    BPW = B // NW
    NQ = H // QROWS

    def body(pr, pb, lr, lb, out,
             lrb0, lrb1, prb0, prb1, pbb0, pbb1, lbb0, lbb1, outv, sb0, sb1):
        wid = lax.axis_index("c") * NSUB + lax.axis_index("s")
        m3 = jnp.where(lax.broadcasted_iota(jnp.int32, (LANES,), 0) >= 8,
                       1.0, 0.0).astype(jnp.float32)

        def iou(tx1, ty1, tx2, ty2, qx1, qy1, qx2, qy2):
            ix1 = jnp.maximum(tx1, qx1)
            iy1 = jnp.maximum(ty1, qy1)
            ix2 = jnp.minimum(tx2, qx2)
            iy2 = jnp.minimum(ty2, qy2)
            inter = jnp.maximum(ix2 - ix1, 0.0) * jnp.maximum(iy2 - iy1, 0.0)
            a1 = (tx2 - tx1) * (ty2 - ty1)
            a2 = (qx2 - qx1) * (qy2 - qy1)
            return inter / (a1 + a2 - inter + 0.0001)

        def corners(x, y, w, h):
            x1 = x - w * 0.5
            y1 = y - h * 0.5
            return x1, y1, x1 + w, y1 + h

        zeros = jnp.zeros((LANES,), jnp.float32)
        neg, pobj, off = zeros, zeros, zeros
        bbufs = ((lrb0, prb0, pbb0, lbb0, sb0), (lrb1, prb1, pbb1, lbb1, sb1))

        for bi in range(BPW):
            b = wid * BPW + bi

            def issue_box(q, slot):
                blr, bpr, bp, bl, sb = bbufs[slot]
                r0 = q * QROWS
                pltpu.async_copy(lr.at[b, :, pl.ds(r0, QROWS), :], blr, sb)
                pltpu.async_copy(pr.at[b, :, pl.ds(r0, QROWS), :], bpr, sb)
                pltpu.async_copy(pb.at[b, :, pl.ds(r0, QROWS), :], bp, sb)
                pltpu.async_copy(lb.at[b, :, pl.ds(r0, QROWS), :], bl, sb)

            def wait_box(slot):
                blr, bpr, bp, bl, sb = bbufs[slot]
                pltpu.make_async_copy(lr.at[b, :, pl.ds(0, QROWS), :], blr, sb).wait()
                pltpu.make_async_copy(pr.at[b, :, pl.ds(0, QROWS), :], bpr, sb).wait()
                pltpu.make_async_copy(pb.at[b, :, pl.ds(0, QROWS), :], bp, sb).wait()
                pltpu.make_async_copy(lb.at[b, :, pl.ds(0, QROWS), :], bl, sb).wait()

            issue_box(0, 0)

            for q in range(NQ):
                slot = q % 2
                if q + 1 < NQ:
                    issue_box(q + 1, 1 - slot)
                wait_box(slot)
                lrb, prb, pbb, lbb = (bbufs[slot][0], bbufs[slot][1],
                                      bbufs[slot][2], bbufs[slot][3])

                def box_body(t, carry, lrb=lrb, prb=prb, pbb=pbb, lbb=lbb):
                    neg, pobj, off = carry
                    y = lax.shift_right_logical(t, 2)
                    j = lax.bitwise_and(t, 3)
                    is_tail = j == 3
                    xoff = jnp.where(is_tail, 40, j * LANES)
                    s = pl.ds(xoff, LANES)
                    mj = jnp.where(is_tail, m3, 1.0).astype(jnp.float32)
                    lr0 = lrb[0, y, s]
                    lr1 = lrb[1, y, s]
                    pr0 = prb[0, y, s]
                    pr1 = prb[1, y, s]
                    valid = jnp.where(lr0 + lr1 > 0.9, mj, 0.0)
                    neg = (neg
                           + (pr0 - lr0) * (pr0 - lr0) * jnp.where(lr0 < 1.0, mj, 0.0)
                           + (pr1 - lr1) * (pr1 - lr1) * jnp.where(lr1 < 1.0, mj, 0.0))
                    lx0, ly0, lw0, lh0 = lbb[0, y, s], lbb[1, y, s], lbb[2, y, s], lbb[3, y, s]
                    lx1, ly1, lw1, lh1 = lbb[4, y, s], lbb[5, y, s], lbb[6, y, s], lbb[7, y, s]
                    px0, py0, pw0, ph0 = pbb[0, y, s], pbb[1, y, s], pbb[2, y, s], pbb[3, y, s]
                    px1, py1, pw1, ph1 = pbb[4, y, s], pbb[5, y, s], pbb[6, y, s], pbb[7, y, s]
                    iou0 = iou(*corners(lx0, ly0, lw0, lh0), *corners(px0, py0, pw0, ph0))
                    iou1 = iou(*corners(lx1, ly1, lw1, lh1), *corners(px1, py1, pw1, ph1))
                    sel = iou1 > iou0
                    best_iou = jnp.where(sel, iou1, iou0)
                    best_pr = jnp.where(sel, pr1, pr0)
                    dr = best_pr - best_iou
                    pobj = pobj + dr * dr * valid
                    ex0, ey0, ew0, eh0 = px0 - lx0, py0 - ly0, pw0 - lw0, ph0 - lh0
                    ex1, ey1, ew1, eh1 = px1 - lx1, py1 - ly1, pw1 - lw1, ph1 - lh1
                    off0 = ex0 * ex0 + ey0 * ey0 + ew0 * ew0 + eh0 * eh0
                    off1 = ex1 * ex1 + ey1 * ey1 + ew1 * ew1 + eh1 * eh1
                    off = off + jnp.where(sel, off1, off0) * valid
                    return neg, pobj, off

                neg, pobj, off = lax.fori_loop(0, QROWS * 4, box_body,
                                               (neg, pobj, off))

        outv[0, :] = pobj
        outv[1, :] = neg
        outv[2, :] = off
        pltpu.sync_copy(outv, out.at[wid])

    return body


def _sc_losses(pred_response, pred_bboxes, label_response, label_bboxes):
    B, BB, H, W = pred_response.shape
    mesh = plsc.VectorSubcoreMesh(core_axis_name="c", subcore_axis_name="s")
    f32 = jnp.float32
    run = pl.kernel(
        _sc_body((B, BB, H, W)),
        out_type=jax.ShapeDtypeStruct((NW, 3, LANES), f32),
        mesh=mesh,
        scratch_types=[
            pltpu.VMEM((BB, QROWS, W), f32),
            pltpu.VMEM((BB, QROWS, W), f32),
            pltpu.VMEM((BB, QROWS, W), f32),
            pltpu.VMEM((BB, QROWS, W), f32),
            pltpu.VMEM((BB * 4, QROWS, W), f32),
            pltpu.VMEM((BB * 4, QROWS, W), f32),
            pltpu.VMEM((BB * 4, QROWS, W), f32),
            pltpu.VMEM((BB * 4, QROWS, W), f32),
            pltpu.VMEM((3, LANES), f32),
            pltpu.SemaphoreType.DMA,
            pltpu.SemaphoreType.DMA,
        ],
    )
    return run(pred_response, pred_bboxes, label_response, label_bboxes)


def _tc_body(pc, lc, lr, out_ref):
    b = pl.program_id(0)
    pc_, lc_ = pc[0], lc[0]
    lr_ = lr[0]
    valid = (lr_[0:1] + lr_[1:2] > 0.9).astype(jnp.float32)
    cls_p = jnp.sum(((pc_ - lc_) ** 2) * valid)
    part = jnp.full((1, 128), cls_p, jnp.float32)

    @pl.when(b == 0)
    def _():
        out_ref[...] = jnp.zeros_like(out_ref)

    out_ref[...] += part


def kernel(pred_cls, pred_response, pred_bboxes, label_cls, label_response, label_bboxes):
    B, CLS, H, W = pred_cls.shape
    BB = pred_response.shape[1]
    HW = H * W

    sc_acc = _sc_losses(pred_response, pred_bboxes, label_response, label_bboxes)

    pc = pred_cls.reshape(B, CLS, HW)
    lc = label_cls.reshape(B, CLS, HW)
    lr = label_response.reshape(B, BB, HW)
    cls_acc = pl.pallas_call(
        _tc_body,
        grid=(B,),
        in_specs=[
            pl.BlockSpec((1, CLS, HW), lambda b: (b, 0, 0)),
            pl.BlockSpec((1, CLS, HW), lambda b: (b, 0, 0)),
            pl.BlockSpec((1, BB, HW), lambda b: (b, 0, 0)),
        ],
        out_specs=pl.BlockSpec((1, 128), lambda b: (0, 0)),
        out_shape=jax.ShapeDtypeStruct((1, 128), jnp.float32),
    )(pc, lc, lr)

    sums = jnp.sum(sc_acc, axis=(0, 2))
    inv_b = 1.0 / B
    return {"pObj": sums[0] * (inv_b * L_OBJ),
            "nObj": sums[1] * (inv_b * L_NOOBJ),
            "cls": cls_acc[0, 0] * inv_b,
            "offset": sums[2] * (inv_b * L_COORD)}


# speedup vs baseline: 1.9606x; 1.0744x over previous
import jax
import jax.numpy as jnp
from jax import lax
from jax.experimental import pallas as pl
from jax.experimental.pallas import tpu as pltpu
from jax.experimental.pallas import tpu_sc as plsc

L_COORD, L_OBJ, L_NOOBJ = 5.0, 1.0, 0.5
NCORE, NSUB, LANES = 2, 16, 16
NW = NCORE * NSUB
QROWS = 8


def _sc_body(shapes):
    B, BB, H, W = shapes
    BPW = B // NW
    NQ = H // QROWS

    def body(pr, pb, lr, lb, out,
             lrb0, lrb1, prb0, prb1, pbb0, pbb1, lbb0, lbb1, outv, sb0, sb1):
        wid = lax.axis_index("c") * NSUB + lax.axis_index("s")
        m3 = jnp.where(lax.broadcasted_iota(jnp.int32, (LANES,), 0) >= 8,
                       1.0, 0.0).astype(jnp.float32)

        def iou(tx1, ty1, tx2, ty2, qx1, qy1, qx2, qy2):
            ix1 = jnp.maximum(tx1, qx1)
            iy1 = jnp.maximum(ty1, qy1)
            ix2 = jnp.minimum(tx2, qx2)
            iy2 = jnp.minimum(ty2, qy2)
            inter = jnp.maximum(ix2 - ix1, 0.0) * jnp.maximum(iy2 - iy1, 0.0)
            a1 = (tx2 - tx1) * (ty2 - ty1)
            a2 = (qx2 - qx1) * (qy2 - qy1)
            return inter / (a1 + a2 - inter + 0.0001)

        def corners(x, y, w, h):
            x1 = x - w * 0.5
            y1 = y - h * 0.5
            return x1, y1, x1 + w, y1 + h

        zeros = jnp.zeros((LANES,), jnp.float32)
        neg, pobj, off = zeros, zeros, zeros
        bbufs = ((lrb0, prb0, pbb0, lbb0, sb0), (lrb1, prb1, pbb1, lbb1, sb1))

        for bi in range(BPW):
            b = wid * BPW + bi

            def issue_box(q, slot):
                blr, bpr, bp, bl, sb = bbufs[slot]
                r0 = q * QROWS
                pltpu.async_copy(lr.at[b, :, pl.ds(r0, QROWS), :], blr, sb)
                pltpu.async_copy(pr.at[b, :, pl.ds(r0, QROWS), :], bpr, sb)
                pltpu.async_copy(pb.at[b, :, pl.ds(r0, QROWS), :], bp, sb)
                pltpu.async_copy(lb.at[b, :, pl.ds(r0, QROWS), :], bl, sb)

            def wait_box(slot):
                blr, bpr, bp, bl, sb = bbufs[slot]
                pltpu.make_async_copy(lr.at[b, :, pl.ds(0, QROWS), :], blr, sb).wait()
                pltpu.make_async_copy(pr.at[b, :, pl.ds(0, QROWS), :], bpr, sb).wait()
                pltpu.make_async_copy(pb.at[b, :, pl.ds(0, QROWS), :], bp, sb).wait()
                pltpu.make_async_copy(lb.at[b, :, pl.ds(0, QROWS), :], bl, sb).wait()

            issue_box(0, 0)

            for q in range(NQ):
                slot = q % 2
                if q + 1 < NQ:
                    issue_box(q + 1, 1 - slot)
                wait_box(slot)
                lrb, prb, pbb, lbb = (bbufs[slot][0], bbufs[slot][1],
                                      bbufs[slot][2], bbufs[slot][3])

                def box_body(t, carry, lrb=lrb, prb=prb, pbb=pbb, lbb=lbb):
                    neg, pobj, off = carry
                    y = lax.shift_right_logical(t, 2)
                    j = lax.bitwise_and(t, 3)
                    is_tail = j == 3
                    xoff = jnp.where(is_tail, 40, j * LANES)
                    s = pl.ds(xoff, LANES)
                    mj = jnp.where(is_tail, m3, 1.0).astype(jnp.float32)
                    lr0 = lrb[0, y, s]
                    lr1 = lrb[1, y, s]
                    pr0 = prb[0, y, s]
                    pr1 = prb[1, y, s]
                    valid = jnp.where(lr0 + lr1 > 0.9, mj, 0.0)
                    neg = (neg
                           + (pr0 - lr0) * (pr0 - lr0) * jnp.where(lr0 < 1.0, mj, 0.0)
                           + (pr1 - lr1) * (pr1 - lr1) * jnp.where(lr1 < 1.0, mj, 0.0))
                    lx0, ly0, lw0, lh0 = lbb[0, y, s], lbb[1, y, s], lbb[2, y, s], lbb[3, y, s]
                    lx1, ly1, lw1, lh1 = lbb[4, y, s], lbb[5, y, s], lbb[6, y, s], lbb[7, y, s]
                    px0, py0, pw0, ph0 = pbb[0, y, s], pbb[1, y, s], pbb[2, y, s], pbb[3, y, s]
                    px1, py1, pw1, ph1 = pbb[4, y, s], pbb[5, y, s], pbb[6, y, s], pbb[7, y, s]
                    iou0 = iou(*corners(lx0, ly0, lw0, lh0), *corners(px0, py0, pw0, ph0))
                    iou1 = iou(*corners(lx1, ly1, lw1, lh1), *corners(px1, py1, pw1, ph1))
                    sel = iou1 > iou0
                    best_iou = jnp.where(sel, iou1, iou0)
                    best_pr = jnp.where(sel, pr1, pr0)
                    dr = best_pr - best_iou
                    pobj = pobj + dr * dr * valid
                    ex0, ey0, ew0, eh0 = px0 - lx0, py0 - ly0, pw0 - lw0, ph0 - lh0
                    ex1, ey1, ew1, eh1 = px1 - lx1, py1 - ly1, pw1 - lw1, ph1 - lh1
                    off0 = ex0 * ex0 + ey0 * ey0 + ew0 * ew0 + eh0 * eh0
                    off1 = ex1 * ex1 + ey1 * ey1 + ew1 * ew1 + eh1 * eh1
                    off = off + jnp.where(sel, off1, off0) * valid
                    return neg, pobj, off

                neg, pobj, off = lax.fori_loop(0, QROWS * 4, box_body,
                                               (neg, pobj, off))

        outv[0, :] = pobj
        outv[1, :] = neg
        outv[2, :] = off
        pltpu.sync_copy(outv, out.at[wid])

    return body


def _sc_losses(pred_response, pred_bboxes, label_response, label_bboxes):
    B, BB, H, W = pred_response.shape
    mesh = plsc.VectorSubcoreMesh(core_axis_name="c", subcore_axis_name="s")
    f32 = jnp.float32
    run = pl.kernel(
        _sc_body((B, BB, H, W)),
        out_type=jax.ShapeDtypeStruct((NW, 3, LANES), f32),
        mesh=mesh,
        scratch_types=[
            pltpu.VMEM((BB, QROWS, W), f32),
            pltpu.VMEM((BB, QROWS, W), f32),
            pltpu.VMEM((BB, QROWS, W), f32),
            pltpu.VMEM((BB, QROWS, W), f32),
            pltpu.VMEM((BB * 4, QROWS, W), f32),
            pltpu.VMEM((BB * 4, QROWS, W), f32),
            pltpu.VMEM((BB * 4, QROWS, W), f32),
            pltpu.VMEM((BB * 4, QROWS, W), f32),
            pltpu.VMEM((3, LANES), f32),
            pltpu.SemaphoreType.DMA,
            pltpu.SemaphoreType.DMA,
        ],
    )
    return run(pred_response, pred_bboxes, label_response, label_bboxes)


def _tc_body(pc, lc, lr, out_ref):
    b = pl.program_id(0)
    cls_p = 0.0
    for i in range(pc.shape[0]):
        valid = (lr[i, 0:1] + lr[i, 1:2] > 0.9).astype(jnp.float32)
        cls_p += jnp.sum(((pc[i] - lc[i]) ** 2) * valid)
    part = jnp.full((1, 128), cls_p, jnp.float32)

    @pl.when(b == 0)
    def _():
        out_ref[...] = jnp.zeros_like(out_ref)

    out_ref[...] += part


def kernel(pred_cls, pred_response, pred_bboxes, label_cls, label_response, label_bboxes):
    B, CLS, H, W = pred_cls.shape
    BB = pred_response.shape[1]
    HW = H * W

    sc_acc = _sc_losses(pred_response, pred_bboxes, label_response, label_bboxes)

    pc = pred_cls.reshape(B, CLS, HW)
    lc = label_cls.reshape(B, CLS, HW)
    lr = label_response.reshape(B, BB, HW)
    BPG = 2
    cls_acc = pl.pallas_call(
        _tc_body,
        grid=(B // BPG,),
        in_specs=[
            pl.BlockSpec((BPG, CLS, HW), lambda b: (b, 0, 0)),
            pl.BlockSpec((BPG, CLS, HW), lambda b: (b, 0, 0)),
            pl.BlockSpec((BPG, BB, HW), lambda b: (b, 0, 0)),
        ],
        out_specs=pl.BlockSpec((1, 128), lambda b: (0, 0)),
        out_shape=jax.ShapeDtypeStruct((1, 128), jnp.float32),
    )(pc, lc, lr)

    sums = jnp.sum(sc_acc, axis=(0, 2))
    inv_b = 1.0 / B
    return {"pObj": sums[0] * (inv_b * L_OBJ),
            "nObj": sums[1] * (inv_b * L_NOOBJ),
            "cls": cls_acc[0, 0] * inv_b,
            "offset": sums[2] * (inv_b * L_COORD)}


# speedup vs baseline: 1.9943x; 1.0172x over previous
import jax
import jax.numpy as jnp
from jax import lax
from jax.experimental import pallas as pl
from jax.experimental.pallas import tpu as pltpu
from jax.experimental.pallas import tpu_sc as plsc

L_COORD, L_OBJ, L_NOOBJ = 5.0, 1.0, 0.5
NCORE, NSUB, LANES = 2, 16, 16
NW = NCORE * NSUB
QROWS = 8


def _sc_body(shapes):
    B, BB, H, W = shapes
    BPW = B // NW
    NQ = H // QROWS

    def body(pr, pb, lr, lb, out,
             lrb0, lrb1, prb0, prb1, pbb0, pbb1, lbb0, lbb1, outv, sb0, sb1):
        wid = lax.axis_index("c") * NSUB + lax.axis_index("s")
        m3 = jnp.where(lax.broadcasted_iota(jnp.int32, (LANES,), 0) >= 8,
                       1.0, 0.0).astype(jnp.float32)

        def iou(tx1, ty1, tx2, ty2, qx1, qy1, qx2, qy2):
            ix1 = jnp.maximum(tx1, qx1)
            iy1 = jnp.maximum(ty1, qy1)
            ix2 = jnp.minimum(tx2, qx2)
            iy2 = jnp.minimum(ty2, qy2)
            inter = jnp.maximum(ix2 - ix1, 0.0) * jnp.maximum(iy2 - iy1, 0.0)
            a1 = (tx2 - tx1) * (ty2 - ty1)
            a2 = (qx2 - qx1) * (qy2 - qy1)
            return inter / (a1 + a2 - inter + 0.0001)

        def corners(x, y, w, h):
            x1 = x - w * 0.5
            y1 = y - h * 0.5
            return x1, y1, x1 + w, y1 + h

        zeros = jnp.zeros((LANES,), jnp.float32)
        neg, pobj, off = zeros, zeros, zeros
        bbufs = ((lrb0, prb0, pbb0, lbb0, sb0), (lrb1, prb1, pbb1, lbb1, sb1))

        for bi in range(BPW):
            b = wid * BPW + bi

            def issue_box(q, slot):
                blr, bpr, bp, bl, sb = bbufs[slot]
                r0 = q * QROWS
                pltpu.async_copy(lr.at[b, :, pl.ds(r0, QROWS), :], blr, sb)
                pltpu.async_copy(pr.at[b, :, pl.ds(r0, QROWS), :], bpr, sb)
                pltpu.async_copy(pb.at[b, :, pl.ds(r0, QROWS), :], bp, sb)
                pltpu.async_copy(lb.at[b, :, pl.ds(r0, QROWS), :], bl, sb)

            def wait_box(slot):
                blr, bpr, bp, bl, sb = bbufs[slot]
                pltpu.make_async_copy(lr.at[b, :, pl.ds(0, QROWS), :], blr, sb).wait()
                pltpu.make_async_copy(pr.at[b, :, pl.ds(0, QROWS), :], bpr, sb).wait()
                pltpu.make_async_copy(pb.at[b, :, pl.ds(0, QROWS), :], bp, sb).wait()
                pltpu.make_async_copy(lb.at[b, :, pl.ds(0, QROWS), :], bl, sb).wait()

            issue_box(0, 0)

            for q in range(NQ):
                slot = q % 2
                if q + 1 < NQ:
                    issue_box(q + 1, 1 - slot)
                wait_box(slot)
                lrb, prb, pbb, lbb = (bbufs[slot][0], bbufs[slot][1],
                                      bbufs[slot][2], bbufs[slot][3])

                def box_body(t, carry, lrb=lrb, prb=prb, pbb=pbb, lbb=lbb):
                    neg, pobj, off = carry
                    y = lax.shift_right_logical(t, 2)
                    j = lax.bitwise_and(t, 3)
                    is_tail = j == 3
                    xoff = jnp.where(is_tail, 40, j * LANES)
                    s = pl.ds(xoff, LANES)
                    mj = jnp.where(is_tail, m3, 1.0).astype(jnp.float32)
                    lr0 = lrb[0, y, s]
                    lr1 = lrb[1, y, s]
                    pr0 = prb[0, y, s]
                    pr1 = prb[1, y, s]
                    valid = jnp.where(lr0 + lr1 > 0.9, mj, 0.0)
                    neg = (neg
                           + (pr0 - lr0) * (pr0 - lr0) * jnp.where(lr0 < 1.0, mj, 0.0)
                           + (pr1 - lr1) * (pr1 - lr1) * jnp.where(lr1 < 1.0, mj, 0.0))
                    lx0, ly0, lw0, lh0 = lbb[0, y, s], lbb[1, y, s], lbb[2, y, s], lbb[3, y, s]
                    lx1, ly1, lw1, lh1 = lbb[4, y, s], lbb[5, y, s], lbb[6, y, s], lbb[7, y, s]
                    px0, py0, pw0, ph0 = pbb[0, y, s], pbb[1, y, s], pbb[2, y, s], pbb[3, y, s]
                    px1, py1, pw1, ph1 = pbb[4, y, s], pbb[5, y, s], pbb[6, y, s], pbb[7, y, s]
                    iou0 = iou(*corners(lx0, ly0, lw0, lh0), *corners(px0, py0, pw0, ph0))
                    iou1 = iou(*corners(lx1, ly1, lw1, lh1), *corners(px1, py1, pw1, ph1))
                    sel = iou1 > iou0
                    best_iou = jnp.where(sel, iou1, iou0)
                    best_pr = jnp.where(sel, pr1, pr0)
                    dr = best_pr - best_iou
                    pobj = pobj + dr * dr * valid
                    ex0, ey0, ew0, eh0 = px0 - lx0, py0 - ly0, pw0 - lw0, ph0 - lh0
                    ex1, ey1, ew1, eh1 = px1 - lx1, py1 - ly1, pw1 - lw1, ph1 - lh1
                    off0 = ex0 * ex0 + ey0 * ey0 + ew0 * ew0 + eh0 * eh0
                    off1 = ex1 * ex1 + ey1 * ey1 + ew1 * ew1 + eh1 * eh1
                    off = off + jnp.where(sel, off1, off0) * valid
                    return neg, pobj, off

                neg, pobj, off = lax.fori_loop(0, QROWS * 4, box_body,
                                               (neg, pobj, off))

        outv[0, :] = pobj
        outv[1, :] = neg
        outv[2, :] = off
        pltpu.sync_copy(outv, out.at[wid])

    return body


def _sc_losses(pred_response, pred_bboxes, label_response, label_bboxes):
    B, BB, H, W = pred_response.shape
    mesh = plsc.VectorSubcoreMesh(core_axis_name="c", subcore_axis_name="s")
    f32 = jnp.float32
    run = pl.kernel(
        _sc_body((B, BB, H, W)),
        out_type=jax.ShapeDtypeStruct((NW, 3, LANES), f32),
        mesh=mesh,
        scratch_types=[
            pltpu.VMEM((BB, QROWS, W), f32),
            pltpu.VMEM((BB, QROWS, W), f32),
            pltpu.VMEM((BB, QROWS, W), f32),
            pltpu.VMEM((BB, QROWS, W), f32),
            pltpu.VMEM((BB * 4, QROWS, W), f32),
            pltpu.VMEM((BB * 4, QROWS, W), f32),
            pltpu.VMEM((BB * 4, QROWS, W), f32),
            pltpu.VMEM((BB * 4, QROWS, W), f32),
            pltpu.VMEM((3, LANES), f32),
            pltpu.SemaphoreType.DMA,
            pltpu.SemaphoreType.DMA,
        ],
    )
    return run(pred_response, pred_bboxes, label_response, label_bboxes)


def _tc_body(pc, lc, lr, out_ref):
    b = pl.program_id(0)
    cls_p = 0.0
    for i in range(pc.shape[0]):
        valid = (lr[i, 0:1] + lr[i, 1:2] > 0.9).astype(jnp.float32)
        cls_p += jnp.sum(((pc[i] - lc[i]) ** 2) * valid)
    part = jnp.full((1, 128), cls_p, jnp.float32)

    @pl.when(b == 0)
    def _():
        out_ref[...] = jnp.zeros_like(out_ref)

    out_ref[...] += part


def kernel(pred_cls, pred_response, pred_bboxes, label_cls, label_response, label_bboxes):
    B, CLS, H, W = pred_cls.shape
    BB = pred_response.shape[1]
    HW = H * W

    sc_acc = _sc_losses(pred_response, pred_bboxes, label_response, label_bboxes)

    pc = pred_cls.reshape(B, CLS, HW)
    lc = label_cls.reshape(B, CLS, HW)
    lr = label_response.reshape(B, BB, HW)
    BPG = 4
    cls_acc = pl.pallas_call(
        _tc_body,
        grid=(B // BPG,),
        in_specs=[
            pl.BlockSpec((BPG, CLS, HW), lambda b: (b, 0, 0)),
            pl.BlockSpec((BPG, CLS, HW), lambda b: (b, 0, 0)),
            pl.BlockSpec((BPG, BB, HW), lambda b: (b, 0, 0)),
        ],
        out_specs=pl.BlockSpec((1, 128), lambda b: (0, 0)),
        out_shape=jax.ShapeDtypeStruct((1, 128), jnp.float32),
    )(pc, lc, lr)

    sums = jnp.sum(sc_acc, axis=(0, 2))
    inv_b = 1.0 / B
    return {"pObj": sums[0] * (inv_b * L_OBJ),
            "nObj": sums[1] * (inv_b * L_NOOBJ),
            "cls": cls_acc[0, 0] * inv_b,
            "offset": sums[2] * (inv_b * L_COORD)}
